# Initial kernel scaffold; baseline (speedup 1.0000x reference)
import jax
import jax.numpy as jnp
from jax.experimental import pallas as pl


def kernel(center, context, negatives, W_center, W_context):
    raise NotImplementedError("write your pallas kernel here")


# speedup vs baseline: 4.8963x; 4.8963x over previous
import functools

import jax
import jax.numpy as jnp
from jax import lax
from jax.experimental import pallas as pl
from jax.experimental.pallas import tpu as pltpu
from jax.experimental.pallas import tpu_sc as plsc

B = 16384
K = 20
D = 64
LANES = 16
NC = 2
NS = 16
NW = NC * NS
BPW = B // NW
G = BPW // LANES


def _dot_rows(crows, xrows, sbuf, ssum):
    strided = lax.iota(jnp.int32, LANES) * LANES

    def body(g, carry):
        rb = g * LANES
        for r in range(LANES):
            acc = (crows[rb + r, pl.ds(0, LANES)]
                   * xrows[rb + r, pl.ds(0, LANES)])
            for q in range(1, D // LANES):
                acc = acc + (crows[rb + r, pl.ds(q * LANES, LANES)]
                             * xrows[rb + r, pl.ds(q * LANES, LANES)])
            ssum[pl.ds(r * LANES, LANES)] = acc
        tot = jnp.zeros((LANES,), jnp.float32)
        for c in range(LANES):
            tot = tot + plsc.load_gather(ssum, [strided + c])
        sbuf[pl.ds(rb, LANES)] = tot
        return carry

    lax.fori_loop(0, G, body, 0)


def _sc_scores_body(center_h, context_h, negT_h, wc_h, wx_h,
                    pos_out, neg_out,
                    cidx, oidx, nidx, crows, xrows, sbuf, ssum, sem):
    wid = lax.axis_index("s") * NC + lax.axis_index("c")
    base = wid * BPW

    pltpu.sync_copy(center_h.at[pl.ds(base, BPW)], cidx)
    pltpu.sync_copy(context_h.at[pl.ds(base, BPW)], oidx)
    for k in range(K):
        pltpu.sync_copy(negT_h.at[pl.ds(k * B + base, BPW)],
                        nidx.at[pl.ds(k * BPW, BPW)])

    pltpu.async_copy(wc_h.at[cidx], crows, sem).wait()
    pltpu.async_copy(wx_h.at[oidx], xrows, sem).wait()
    _dot_rows(crows, xrows, sbuf, ssum)
    pltpu.sync_copy(sbuf, pos_out.at[pl.ds(base, BPW)])

    def kbody(k, carry):
        pltpu.async_copy(wx_h.at[nidx.at[pl.ds(k * BPW, BPW)]],
                         xrows, sem).wait()
        _dot_rows(crows, xrows, sbuf, ssum)
        pltpu.sync_copy(sbuf, neg_out.at[pl.ds(k * B + base, BPW)])
        return carry

    lax.fori_loop(0, K, kbody, 0)


_sc_scores = functools.partial(
    pl.kernel,
    out_type=[jax.ShapeDtypeStruct((B,), jnp.float32),
              jax.ShapeDtypeStruct((K * B,), jnp.float32)],
    mesh=plsc.VectorSubcoreMesh(core_axis_name="c", subcore_axis_name="s"),
    compiler_params=pltpu.CompilerParams(
        needs_layout_passes=False, use_tc_tiling_on_sc=False),
    scratch_types=[
        pltpu.VMEM((BPW,), jnp.int32),
        pltpu.VMEM((BPW,), jnp.int32),
        pltpu.VMEM((K * BPW,), jnp.int32),
        pltpu.VMEM((BPW, D), jnp.float32),
        pltpu.VMEM((BPW, D), jnp.float32),
        pltpu.VMEM((BPW,), jnp.float32),
        pltpu.VMEM((LANES * LANES,), jnp.float32),
        pltpu.SemaphoreType.DMA,
    ],
)(_sc_scores_body)


def _loss_body(pos_ref, neg_ref, out_ref):
    pos = pos_ref[...]
    neg = neg_ref[...]
    s = jnp.sum(jax.nn.log_sigmoid(pos)) + jnp.sum(jax.nn.log_sigmoid(-neg))
    out_ref[0, 0] = -s / B


def _tc_loss(pos2d, neg2d):
    return pl.pallas_call(
        _loss_body,
        out_shape=jax.ShapeDtypeStruct((1, 1), jnp.float32),
        out_specs=pl.BlockSpec(memory_space=pltpu.SMEM),
    )(pos2d, neg2d)


def kernel(center, context, negatives, W_center, W_context):
    center = center.astype(jnp.int32)
    context = context.astype(jnp.int32)
    negT = negatives.astype(jnp.int32).T.reshape(K * B)
    pos, negs = _sc_scores(center, context, negT, W_center, W_context)
    loss = _tc_loss(pos.reshape(B // 128, 128), negs.reshape(K * B // 128, 128))
    return loss[0, 0]
